# Initial kernel scaffold; baseline (speedup 1.0000x reference)
import jax
import jax.numpy as jnp
from jax.experimental import pallas as pl


def kernel(h, edge_index, edge_attr, Wm1, bm1, Wm2, bm2, Wu1, bu1, Wu2, bu2, gn, bn, We1, be1, We2, be2, Wg, bg, ge, be):
    raise NotImplementedError("write your pallas kernel here")


# speedup vs baseline: 1.9554x; 1.9554x over previous
import functools

import jax
import jax.numpy as jnp
from jax import lax
from jax.experimental import pallas as pl
from jax.experimental.pallas import tpu as pltpu
from jax.experimental.pallas import tpu_sc as plsc

HIDDEN = 128
EDGE_DIM = 16
EDGE_SCALE = 0.1
_NW = 32
_SQRT1_2 = 0.7071067811865476


def _gelu(x):
    return 0.5 * x * (1.0 + lax.erf(x * _SQRT1_2))


def _sc_gather(h, idx_flat):
    n_idx = idx_flat.shape[0]
    per_w = n_idx // _NW
    ch = 80
    n_ch = per_w // ch
    mesh = plsc.VectorSubcoreMesh(core_axis_name="c", subcore_axis_name="s")

    @functools.partial(
        pl.kernel,
        out_type=jax.ShapeDtypeStruct((n_idx, HIDDEN), jnp.float32),
        mesh=mesh,
        scratch_types=[
            pltpu.VMEM((ch,), jnp.int32),
            pltpu.VMEM((ch, HIDDEN), jnp.float32),
            pltpu.SemaphoreType.DMA,
        ],
    )
    def k(h_hbm, idx_hbm, out_hbm, idx_v, rows_v, sem):
        c = lax.axis_index("c")
        s = lax.axis_index("s")
        base = (c * 16 + s) * per_w

        def body(g, carry):
            off = base + g * ch
            pltpu.sync_copy(idx_hbm.at[pl.ds(off, ch)], idx_v)
            pltpu.async_copy(h_hbm.at[idx_v], rows_v, sem).wait()
            pltpu.sync_copy(rows_v, out_hbm.at[pl.ds(off, ch)])
            return carry

        lax.fori_loop(0, n_ch, body, 0)

    return k(h, idx_flat)


def _sc_scatter(m, dst, n_nodes):
    n_edges = m.shape[0]
    per_w = n_edges // _NW
    ch = 80
    n_ch = per_w // ch
    n_rch = n_nodes // ch
    mesh = plsc.VectorSubcoreMesh(core_axis_name="c", subcore_axis_name="s")

    @functools.partial(
        pl.kernel,
        out_type=jax.ShapeDtypeStruct((2 * n_nodes, HIDDEN), jnp.float32),
        mesh=mesh,
        scratch_types=[
            pltpu.VMEM((ch,), jnp.int32),
            pltpu.VMEM((ch, HIDDEN), jnp.float32),
            pltpu.VMEM((ch, HIDDEN), jnp.float32),
            pltpu.VMEM_SHARED((n_nodes, HIDDEN), jnp.float32),
            pltpu.SemaphoreType.DMA,
        ],
    )
    def k(m_hbm, dst_hbm, out_hbm, idx_v, m_v, zbuf, acc_sh, sem):
        c = lax.axis_index("c")
        s = lax.axis_index("s")

        zero16 = jnp.zeros((16,), jnp.float32)

        def zrow(i, carry):
            for j in range(HIDDEN // 16):
                zbuf[i, pl.ds(j * 16, 16)] = zero16
            return carry

        lax.fori_loop(0, ch, zrow, 0)

        n_mine = (n_rch - s + 15) // 16

        def zchunk(k_, carry):
            cid = s + k_ * 16
            pltpu.sync_copy(zbuf, acc_sh.at[pl.ds(cid * ch, ch)])
            return carry

        lax.fori_loop(0, n_mine, zchunk, 0)
        plsc.subcore_barrier()

        base = (c * 16 + s) * per_w

        def body(g, carry):
            off = base + g * ch
            pltpu.sync_copy(dst_hbm.at[pl.ds(off, ch)], idx_v)
            pltpu.sync_copy(m_hbm.at[pl.ds(off, ch)], m_v)
            pltpu.sync_copy(m_v, acc_sh.at[idx_v], add=True)
            return carry

        lax.fori_loop(0, n_ch, body, 0)
        plsc.subcore_barrier()

        def wchunk(k_, carry):
            r = (s + k_ * 16) * ch
            pltpu.sync_copy(acc_sh.at[pl.ds(r, ch)], zbuf)
            pltpu.sync_copy(zbuf, out_hbm.at[pl.ds(c * n_nodes + r, ch)])
            return carry

        lax.fori_loop(0, n_mine, wchunk, 0)

    return k(m, dst)


def _tc_edge(gathered, edge_attr, We1a, We1b, We1c, be1, We2, be2,
             Wga, Wgb, Wgc, bg, ge, be, Wm1a, Wm1b, bm1, Wm2, bm2):
    n_edges = edge_attr.shape[0]
    blk = 512
    nb = n_edges // blk

    def dot(a, b):
        return lax.dot_general(a, b, (((1,), (0,)), ((), ())),
                               preferred_element_type=jnp.float32)

    def body(hs_r, hd_r, ea_r, We1a_r, We1b_r, We1c_r, be1_r, We2_r, be2_r,
             Wga_r, Wgb_r, Wgc_r, bg_r, ge_r, be_r, Wm1a_r, Wm1b_r, bm1_r,
             Wm2_r, bm2_r, e_ref, m_ref):
        hs = hs_r[...]
        hd = hd_r[...]
        ea = ea_r[...]
        g = dot(hs, Wga_r[...]) + dot(hd, Wgb_r[...]) + dot(ea, Wgc_r[...]) + bg_r[...]
        gate = jax.nn.sigmoid(g)
        t = dot(hs, We1a_r[...]) + dot(hd, We1b_r[...]) + dot(ea, We1c_r[...]) + be1_r[...]
        t = _gelu(t)
        delta = (dot(t, We2_r[...]) + be2_r[...]) * gate
        x = ea + EDGE_SCALE * delta
        mu = jnp.mean(x, axis=-1, keepdims=True)
        var = jnp.mean((x - mu) ** 2, axis=-1, keepdims=True)
        e = (x - mu) / jnp.sqrt(var + 1e-5) * ge_r[...] + be_r[...]
        e_ref[...] = e
        u = _gelu(dot(hs, Wm1a_r[...]) + dot(e, Wm1b_r[...]) + bm1_r[...])
        m_ref[...] = dot(u, Wm2_r[...]) + bm2_r[...]

    wspec = lambda shp: pl.BlockSpec(shp, lambda i: (0, 0))
    return pl.pallas_call(
        body,
        grid=(nb,),
        in_specs=[
            pl.BlockSpec((blk, HIDDEN), lambda i: (i, 0)),
            pl.BlockSpec((blk, HIDDEN), lambda i: (i + nb, 0)),
            pl.BlockSpec((blk, EDGE_DIM), lambda i: (i, 0)),
            wspec((HIDDEN, EDGE_DIM)), wspec((HIDDEN, EDGE_DIM)),
            wspec((EDGE_DIM, EDGE_DIM)), wspec((1, EDGE_DIM)),
            wspec((EDGE_DIM, EDGE_DIM)), wspec((1, EDGE_DIM)),
            wspec((HIDDEN, 1)), wspec((HIDDEN, 1)), wspec((EDGE_DIM, 1)),
            wspec((1, 1)),
            wspec((1, EDGE_DIM)), wspec((1, EDGE_DIM)),
            wspec((HIDDEN, HIDDEN)), wspec((EDGE_DIM, HIDDEN)),
            wspec((1, HIDDEN)),
            wspec((HIDDEN, HIDDEN)), wspec((1, HIDDEN)),
        ],
        out_specs=[
            pl.BlockSpec((blk, EDGE_DIM), lambda i: (i, 0)),
            pl.BlockSpec((blk, HIDDEN), lambda i: (i, 0)),
        ],
        out_shape=[
            jax.ShapeDtypeStruct((n_edges, EDGE_DIM), jnp.float32),
            jax.ShapeDtypeStruct((n_edges, HIDDEN), jnp.float32),
        ],
    )(gathered, gathered, edge_attr, We1a, We1b, We1c, be1, We2, be2,
      Wga, Wgb, Wgc, bg, ge, be, Wm1a, Wm1b, bm1, Wm2, bm2)


def _tc_node(h, parts, Wu1a, Wu1b, bu1, Wu2, bu2, gn, bn):
    n_nodes = h.shape[0]
    blk = 1000
    nb = n_nodes // blk

    def dot(a, b):
        return lax.dot_general(a, b, (((1,), (0,)), ((), ())),
                               preferred_element_type=jnp.float32)

    def body(h_r, p0_r, p1_r, Wu1a_r, Wu1b_r, bu1_r, Wu2_r, bu2_r, gn_r, bn_r,
             o_ref):
        hh = h_r[...]
        agg = p0_r[...] + p1_r[...]
        u = _gelu(dot(hh, Wu1a_r[...]) + dot(agg, Wu1b_r[...]) + bu1_r[...])
        h2 = dot(u, Wu2_r[...]) + bu2_r[...]
        x = hh + h2
        mu = jnp.mean(x, axis=-1, keepdims=True)
        var = jnp.mean((x - mu) ** 2, axis=-1, keepdims=True)
        o_ref[...] = (x - mu) / jnp.sqrt(var + 1e-5) * gn_r[...] + bn_r[...]

    wspec = lambda shp: pl.BlockSpec(shp, lambda i: (0, 0))
    return pl.pallas_call(
        body,
        grid=(nb,),
        in_specs=[
            pl.BlockSpec((blk, HIDDEN), lambda i: (i, 0)),
            pl.BlockSpec((blk, HIDDEN), lambda i: (i, 0)),
            pl.BlockSpec((blk, HIDDEN), lambda i: (i + nb, 0)),
            wspec((HIDDEN, HIDDEN)), wspec((HIDDEN, HIDDEN)),
            wspec((1, HIDDEN)),
            wspec((HIDDEN, HIDDEN)), wspec((1, HIDDEN)),
            wspec((1, HIDDEN)), wspec((1, HIDDEN)),
        ],
        out_specs=pl.BlockSpec((blk, HIDDEN), lambda i: (i, 0)),
        out_shape=jax.ShapeDtypeStruct((n_nodes, HIDDEN), jnp.float32),
    )(h, parts, parts, Wu1a, Wu1b, bu1, Wu2, bu2, gn, bn)


def kernel(h, edge_index, edge_attr, Wm1, bm1, Wm2, bm2, Wu1, bu1, Wu2, bu2,
           gn, bn, We1, be1, We2, be2, Wg, bg, ge, be):
    n_nodes = h.shape[0]
    H, D = HIDDEN, EDGE_DIM

    idx_flat = edge_index.reshape(-1)
    dst = edge_index[1]

    row = lambda v: v.reshape(1, -1)

    gathered = _sc_gather(h, idx_flat)
    e, m = _tc_edge(
        gathered, edge_attr,
        We1[:H], We1[H:2 * H], We1[2 * H:], row(be1), We2, row(be2),
        Wg[:H], Wg[H:2 * H], Wg[2 * H:], row(bg), row(ge), row(be),
        Wm1[:H], Wm1[H:], row(bm1), Wm2, row(bm2))
    parts = _sc_scatter(m, dst, n_nodes)
    hn = _tc_node(h, parts, Wu1[:H], Wu1[H:], row(bu1), Wu2, row(bu2),
                  row(gn), row(bn))
    return (hn, e)


# speedup vs baseline: 2.3952x; 1.2249x over previous
import functools

import jax
import jax.numpy as jnp
from jax import lax
from jax.experimental import pallas as pl
from jax.experimental.pallas import tpu as pltpu
from jax.experimental.pallas import tpu_sc as plsc

HIDDEN = 128
EDGE_DIM = 16
EDGE_SCALE = 0.1
_NW = 32
_SQRT1_2 = 0.7071067811865476


def _gelu(x):
    return 0.5 * x * (1.0 + lax.erf(x * _SQRT1_2))


def _sc_gather(h, idx_flat):
    n_idx = idx_flat.shape[0]
    per_w = n_idx // _NW
    ch = 80
    n_ch = per_w // ch
    mesh = plsc.VectorSubcoreMesh(core_axis_name="c", subcore_axis_name="s")

    @functools.partial(
        pl.kernel,
        out_type=jax.ShapeDtypeStruct((n_idx, HIDDEN), jnp.float32),
        mesh=mesh,
        scratch_types=[
            pltpu.VMEM((ch,), jnp.int32),
            pltpu.VMEM((ch, HIDDEN), jnp.float32),
            pltpu.SemaphoreType.DMA,
        ],
    )
    def k(h_hbm, idx_hbm, out_hbm, idx_v, rows_v, sem):
        c = lax.axis_index("c")
        s = lax.axis_index("s")
        base = (c * 16 + s) * per_w

        def body(g, carry):
            off = base + g * ch
            pltpu.sync_copy(idx_hbm.at[pl.ds(off, ch)], idx_v)
            pltpu.async_copy(h_hbm.at[idx_v], rows_v, sem).wait()
            pltpu.sync_copy(rows_v, out_hbm.at[pl.ds(off, ch)])
            return carry

        lax.fori_loop(0, n_ch, body, 0)

    return k(h, idx_flat)


def _sc_scatter(m, dst, n_nodes):
    n_edges = m.shape[0]
    per_w = n_edges // _NW
    ch = 80
    n_ch = per_w // ch
    n_rch = n_nodes // ch
    mesh = plsc.VectorSubcoreMesh(core_axis_name="c", subcore_axis_name="s")

    @functools.partial(
        pl.kernel,
        out_type=jax.ShapeDtypeStruct((2 * n_nodes, HIDDEN), jnp.float32),
        mesh=mesh,
        scratch_types=[
            pltpu.VMEM((ch,), jnp.int32),
            pltpu.VMEM((ch, HIDDEN), jnp.float32),
            pltpu.VMEM((ch, HIDDEN), jnp.float32),
            pltpu.VMEM_SHARED((n_nodes, HIDDEN), jnp.float32),
            pltpu.SemaphoreType.DMA,
        ],
    )
    def k(m_hbm, dst_hbm, out_hbm, idx_v, m_v, zbuf, acc_sh, sem):
        c = lax.axis_index("c")
        s = lax.axis_index("s")

        zero16 = jnp.zeros((16,), jnp.float32)

        def zrow(i, carry):
            for j in range(HIDDEN // 16):
                zbuf[i, pl.ds(j * 16, 16)] = zero16
            return carry

        lax.fori_loop(0, ch, zrow, 0)

        n_mine = (n_rch - s + 15) // 16

        def zchunk(k_, carry):
            cid = s + k_ * 16
            pltpu.sync_copy(zbuf, acc_sh.at[pl.ds(cid * ch, ch)])
            return carry

        lax.fori_loop(0, n_mine, zchunk, 0)
        plsc.subcore_barrier()

        base = (c * 16 + s) * per_w

        def body(g, carry):
            off = base + g * ch
            pltpu.sync_copy(dst_hbm.at[pl.ds(off, ch)], idx_v)
            pltpu.sync_copy(m_hbm.at[pl.ds(off, ch)], m_v)
            pltpu.sync_copy(m_v, acc_sh.at[idx_v], add=True)
            return carry

        lax.fori_loop(0, n_ch, body, 0)
        plsc.subcore_barrier()

        def wchunk(k_, carry):
            r = (s + k_ * 16) * ch
            pltpu.sync_copy(acc_sh.at[pl.ds(r, ch)], zbuf)
            pltpu.sync_copy(zbuf, out_hbm.at[pl.ds(c * n_nodes + r, ch)])
            return carry

        lax.fori_loop(0, n_mine, wchunk, 0)

    return k(m, dst)


def _tc_edge(gathered, edge_attr, Ws, Wd, Wa, be1, We2, be2, bg, ge, be,
             Wm1b, bm1, Wm2, bm2):
    n_edges = edge_attr.shape[0]
    blk = 1280
    nb = n_edges // blk
    bf = jnp.bfloat16

    def dot(a, b):
        return lax.dot_general(a, b, (((1,), (0,)), ((), ())),
                               preferred_element_type=jnp.float32)

    def body(hs_r, hd_r, ea_r, Ws_r, Wd_r, Wa_r, be1_r, We2_r, be2_r,
             bg_r, ge_r, be_r, Wm1b_r, bm1_r, Wm2_r, bm2_r, e_ref, m_ref):
        hs = hs_r[...].astype(bf)
        hd = hd_r[...].astype(bf)
        ea = ea_r[...]
        y_s = dot(hs, Ws_r[...])
        y_d = dot(hd, Wd_r[...])
        y_a = dot(ea.astype(bf), Wa_r[...])
        t = _gelu(y_s[:, HIDDEN:HIDDEN + EDGE_DIM]
                  + y_d[:, :EDGE_DIM] + y_a[:, :EDGE_DIM] + be1_r[...])
        g = (y_s[:, HIDDEN + EDGE_DIM:] + y_d[:, EDGE_DIM:]
             + y_a[:, EDGE_DIM:] + bg_r[...])
        gate = jax.nn.sigmoid(g)
        delta = (dot(t.astype(bf), We2_r[...]) + be2_r[...]) * gate
        x = ea + EDGE_SCALE * delta
        mu = jnp.mean(x, axis=-1, keepdims=True)
        var = jnp.mean((x - mu) ** 2, axis=-1, keepdims=True)
        e = (x - mu) / jnp.sqrt(var + 1e-5) * ge_r[...] + be_r[...]
        e_ref[...] = e
        u = _gelu(y_s[:, :HIDDEN] + dot(e.astype(bf), Wm1b_r[...]) + bm1_r[...])
        m_ref[...] = dot(u.astype(bf), Wm2_r[...]) + bm2_r[...]

    wspec = lambda shp: pl.BlockSpec(shp, lambda i: (0, 0))
    return pl.pallas_call(
        body,
        grid=(nb,),
        in_specs=[
            pl.BlockSpec((blk, HIDDEN), lambda i: (i, 0)),
            pl.BlockSpec((blk, HIDDEN), lambda i: (i + nb, 0)),
            pl.BlockSpec((blk, EDGE_DIM), lambda i: (i, 0)),
            wspec((HIDDEN, 145)), wspec((HIDDEN, 17)), wspec((EDGE_DIM, 17)),
            wspec((1, EDGE_DIM)),
            wspec((EDGE_DIM, EDGE_DIM)), wspec((1, EDGE_DIM)),
            wspec((1, 1)),
            wspec((1, EDGE_DIM)), wspec((1, EDGE_DIM)),
            wspec((EDGE_DIM, HIDDEN)), wspec((1, HIDDEN)),
            wspec((HIDDEN, HIDDEN)), wspec((1, HIDDEN)),
        ],
        out_specs=[
            pl.BlockSpec((blk, EDGE_DIM), lambda i: (i, 0)),
            pl.BlockSpec((blk, HIDDEN), lambda i: (i, 0)),
        ],
        out_shape=[
            jax.ShapeDtypeStruct((n_edges, EDGE_DIM), jnp.float32),
            jax.ShapeDtypeStruct((n_edges, HIDDEN), jnp.float32),
        ],
    )(gathered, gathered, edge_attr, Ws, Wd, Wa, be1, We2, be2,
      bg, ge, be, Wm1b, bm1, Wm2, bm2)


def _tc_node(h, parts, Wu1a, Wu1b, bu1, Wu2, bu2, gn, bn):
    n_nodes = h.shape[0]
    blk = 1000
    nb = n_nodes // blk

    def dot(a, b):
        return lax.dot_general(a, b, (((1,), (0,)), ((), ())),
                               preferred_element_type=jnp.float32)

    bf = jnp.bfloat16

    def body(h_r, p0_r, p1_r, Wu1a_r, Wu1b_r, bu1_r, Wu2_r, bu2_r, gn_r, bn_r,
             o_ref):
        hh = h_r[...]
        agg = p0_r[...] + p1_r[...]
        u = _gelu(dot(hh.astype(bf), Wu1a_r[...])
                  + dot(agg.astype(bf), Wu1b_r[...]) + bu1_r[...])
        h2 = dot(u.astype(bf), Wu2_r[...]) + bu2_r[...]
        x = hh + h2
        mu = jnp.mean(x, axis=-1, keepdims=True)
        var = jnp.mean((x - mu) ** 2, axis=-1, keepdims=True)
        o_ref[...] = (x - mu) / jnp.sqrt(var + 1e-5) * gn_r[...] + bn_r[...]

    wspec = lambda shp: pl.BlockSpec(shp, lambda i: (0, 0))
    return pl.pallas_call(
        body,
        grid=(nb,),
        in_specs=[
            pl.BlockSpec((blk, HIDDEN), lambda i: (i, 0)),
            pl.BlockSpec((blk, HIDDEN), lambda i: (i, 0)),
            pl.BlockSpec((blk, HIDDEN), lambda i: (i + nb, 0)),
            wspec((HIDDEN, HIDDEN)), wspec((HIDDEN, HIDDEN)),
            wspec((1, HIDDEN)),
            wspec((HIDDEN, HIDDEN)), wspec((1, HIDDEN)),
            wspec((1, HIDDEN)), wspec((1, HIDDEN)),
        ],
        out_specs=pl.BlockSpec((blk, HIDDEN), lambda i: (i, 0)),
        out_shape=jax.ShapeDtypeStruct((n_nodes, HIDDEN), jnp.float32),
    )(h, parts, parts, Wu1a, Wu1b, bu1, Wu2, bu2, gn, bn)


def kernel(h, edge_index, edge_attr, Wm1, bm1, Wm2, bm2, Wu1, bu1, Wu2, bu2,
           gn, bn, We1, be1, We2, be2, Wg, bg, ge, be):
    n_nodes = h.shape[0]
    H, D = HIDDEN, EDGE_DIM

    idx_flat = edge_index.reshape(-1)
    dst = edge_index[1]

    row = lambda v: v.reshape(1, -1)
    bf = jnp.bfloat16

    Ws = jnp.concatenate([Wm1[:H], We1[:H], Wg[:H]], axis=1).astype(bf)
    Wd = jnp.concatenate([We1[H:2 * H], Wg[H:2 * H]], axis=1).astype(bf)
    Wa = jnp.concatenate([We1[2 * H:], Wg[2 * H:]], axis=1).astype(bf)

    gathered = _sc_gather(h, idx_flat)
    e, m = _tc_edge(
        gathered, edge_attr, Ws, Wd, Wa, row(be1), We2.astype(bf), row(be2),
        row(bg), row(ge), row(be), Wm1[H:].astype(bf), row(bm1),
        Wm2.astype(bf), row(bm2))
    parts = _sc_scatter(m, dst, n_nodes)
    hn = _tc_node(h, parts, Wu1[:H].astype(bf), Wu1[H:].astype(bf), row(bu1),
                  Wu2.astype(bf), row(bu2), row(gn), row(bn))
    return (hn, e)


# speedup vs baseline: 3.2783x; 1.3687x over previous
import functools

import jax
import jax.numpy as jnp
from jax import lax
from jax.experimental import pallas as pl
from jax.experimental.pallas import tpu as pltpu
from jax.experimental.pallas import tpu_sc as plsc

HIDDEN = 128
EDGE_DIM = 16
EDGE_SCALE = 0.1
_NW = 32
_SQRT1_2 = 0.7071067811865476


def _gelu(x):
    return 0.5 * x * (1.0 + lax.erf(x * _SQRT1_2))


def _sc_gather(h, idx_flat):
    n_idx = idx_flat.shape[0]
    per_w = n_idx // _NW
    ch = 80
    n_ch = per_w // ch
    n_pair = n_ch // 2
    mesh = plsc.VectorSubcoreMesh(core_axis_name="c", subcore_axis_name="s")

    @functools.partial(
        pl.kernel,
        out_type=jax.ShapeDtypeStruct((n_idx, HIDDEN), jnp.float32),
        mesh=mesh,
        scratch_types=[
            pltpu.VMEM((per_w,), jnp.int32),
            pltpu.VMEM((ch, HIDDEN), jnp.float32),
            pltpu.VMEM((ch, HIDDEN), jnp.float32),
            pltpu.SemaphoreType.DMA,
            pltpu.SemaphoreType.DMA,
            pltpu.SemaphoreType.DMA,
            pltpu.SemaphoreType.DMA,
        ],
    )
    def k(h_hbm, idx_hbm, out_hbm, idx_all, rows0, rows1, gs0, gs1, ws0, ws1):
        c = lax.axis_index("c")
        s = lax.axis_index("s")
        base = (c * 16 + s) * per_w
        pltpu.sync_copy(idx_hbm.at[pl.ds(base, per_w)], idx_all)

        def gat(g, rows, sem):
            pltpu.async_copy(h_hbm.at[idx_all.at[pl.ds(g * ch, ch)]],
                             rows, sem)

        def wr(g, rows, sem):
            pltpu.async_copy(rows, out_hbm.at[pl.ds(base + g * ch, ch)], sem)

        def wr_wait(g, rows, sem):
            pltpu.make_async_copy(
                rows, out_hbm.at[pl.ds(base + g * ch, ch)], sem).wait()

        def g_wait(g, rows, sem):
            pltpu.make_async_copy(h_hbm.at[idx_all.at[pl.ds(g * ch, ch)]],
                                  rows, sem).wait()

        gat(0, rows0, gs0)

        def body(p, carry):
            c0 = 2 * p
            c1 = c0 + 1

            @pl.when(p > 0)
            def _():
                wr_wait(c1 - 2, rows1, ws1)

            gat(c1, rows1, gs1)
            g_wait(c0, rows0, gs0)
            wr(c0, rows0, ws0)
            g_wait(c1, rows1, gs1)
            wr(c1, rows1, ws1)

            @pl.when(p < n_pair - 1)
            def _():
                wr_wait(c0, rows0, ws0)
                gat(c0 + 2, rows0, gs0)

            return carry

        lax.fori_loop(0, n_pair, body, 0)
        wr_wait(n_ch - 2, rows0, ws0)
        wr_wait(n_ch - 1, rows1, ws1)

    return k(h, idx_flat)


def _sc_scatter(m, dst, n_nodes):
    n_edges = m.shape[0]
    per_w = n_edges // _NW
    ch = 80
    n_ch = per_w // ch
    n_rch = n_nodes // ch
    mesh = plsc.VectorSubcoreMesh(core_axis_name="c", subcore_axis_name="s")

    n_pair = n_ch // 2

    @functools.partial(
        pl.kernel,
        out_type=jax.ShapeDtypeStruct((2 * n_nodes, HIDDEN), jnp.float32),
        mesh=mesh,
        scratch_types=[
            pltpu.VMEM((ch,), jnp.int32),
            pltpu.VMEM((ch,), jnp.int32),
            pltpu.VMEM((ch, HIDDEN), jnp.float32),
            pltpu.VMEM((ch, HIDDEN), jnp.float32),
            pltpu.VMEM_SHARED((n_nodes, HIDDEN), jnp.float32),
            pltpu.SemaphoreType.DMA,
            pltpu.SemaphoreType.DMA,
            pltpu.SemaphoreType.DMA,
            pltpu.SemaphoreType.DMA,
        ],
    )
    def k(m_hbm, dst_hbm, out_hbm, i0, i1, m0, m1, acc_sh, ls0, ls1, ss0, ss1):
        c = lax.axis_index("c")
        s = lax.axis_index("s")

        zero16 = jnp.zeros((16,), jnp.float32)

        def zrow(i, carry):
            for j in range(HIDDEN // 16):
                m0[i, pl.ds(j * 16, 16)] = zero16
            return carry

        lax.fori_loop(0, ch, zrow, 0)

        n_mine = (n_rch - s + 15) // 16

        def zchunk(k_, carry):
            cid = s + k_ * 16
            pltpu.sync_copy(m0, acc_sh.at[pl.ds(cid * ch, ch)])
            return carry

        lax.fori_loop(0, n_mine, zchunk, 0)

        base = (c * 16 + s) * per_w
        plsc.subcore_barrier()

        def ld(g, buf, sem):
            pltpu.async_copy(m_hbm.at[pl.ds(base + g * ch, ch)], buf, sem)

        def ld_wait(g, buf, sem):
            pltpu.make_async_copy(m_hbm.at[pl.ds(base + g * ch, ch)], buf,
                                  sem).wait()

        def ldi(g, ibuf):
            pltpu.sync_copy(dst_hbm.at[pl.ds(base + g * ch, ch)], ibuf)

        def sc(ibuf, buf, sem):
            pltpu.async_copy(buf, acc_sh.at[ibuf], sem, add=True)

        def sc_wait(ibuf, buf, sem):
            pltpu.make_async_copy(buf, acc_sh.at[ibuf], sem).wait()

        ldi(0, i0)
        ld(0, m0, ls0)

        def body(p, carry):
            c0 = 2 * p
            c1 = c0 + 1
            ld(c1, m1, ls1)
            ldi(c1, i1)
            ld_wait(c0, m0, ls0)
            pltpu.sync_copy(m0, acc_sh.at[i0], add=True)

            @pl.when(p < n_pair - 1)
            def _():
                ld(c0 + 2, m0, ls0)
                ldi(c0 + 2, i0)

            ld_wait(c1, m1, ls1)
            pltpu.sync_copy(m1, acc_sh.at[i1], add=True)
            return carry

        lax.fori_loop(0, n_pair, body, 0)
        plsc.subcore_barrier()

        def wchunk(k_, carry):
            r = (s + k_ * 16) * ch
            pltpu.sync_copy(acc_sh.at[pl.ds(r, ch)], m0)
            pltpu.sync_copy(m0, out_hbm.at[pl.ds(c * n_nodes + r, ch)])
            return carry

        lax.fori_loop(0, n_mine, wchunk, 0)

    return k(m, dst)


def _tc_edge(gathered, edge_attr, Ws, Wd, Wa, be1, We2, be2, bg, ge, be,
             Wm1b, bm1, Wm2, bm2):
    n_edges = edge_attr.shape[0]
    blk = 1280
    nb = n_edges // blk
    bf = jnp.bfloat16

    def dot(a, b):
        return lax.dot_general(a, b, (((1,), (0,)), ((), ())),
                               preferred_element_type=jnp.float32)

    def body(hs_r, hd_r, ea_r, Ws_r, Wd_r, Wa_r, be1_r, We2_r, be2_r,
             bg_r, ge_r, be_r, Wm1b_r, bm1_r, Wm2_r, bm2_r, e_ref, m_ref):
        hs = hs_r[...].astype(bf)
        hd = hd_r[...].astype(bf)
        ea = ea_r[...]
        y_s = dot(hs, Ws_r[...])
        y_d = dot(hd, Wd_r[...])
        y_a = dot(ea.astype(bf), Wa_r[...])
        t = _gelu(y_s[:, HIDDEN:HIDDEN + EDGE_DIM]
                  + y_d[:, :EDGE_DIM] + y_a[:, :EDGE_DIM] + be1_r[...])
        g = (y_s[:, HIDDEN + EDGE_DIM:] + y_d[:, EDGE_DIM:]
             + y_a[:, EDGE_DIM:] + bg_r[...])
        gate = jax.nn.sigmoid(g)
        delta = (dot(t.astype(bf), We2_r[...]) + be2_r[...]) * gate
        x = ea + EDGE_SCALE * delta
        mu = jnp.mean(x, axis=-1, keepdims=True)
        var = jnp.mean((x - mu) ** 2, axis=-1, keepdims=True)
        e = (x - mu) / jnp.sqrt(var + 1e-5) * ge_r[...] + be_r[...]
        e_ref[...] = e
        u = _gelu(y_s[:, :HIDDEN] + dot(e.astype(bf), Wm1b_r[...]) + bm1_r[...])
        m_ref[...] = dot(u.astype(bf), Wm2_r[...]) + bm2_r[...]

    wspec = lambda shp: pl.BlockSpec(shp, lambda i: (0, 0))
    return pl.pallas_call(
        body,
        grid=(nb,),
        in_specs=[
            pl.BlockSpec((blk, HIDDEN), lambda i: (i, 0)),
            pl.BlockSpec((blk, HIDDEN), lambda i: (i + nb, 0)),
            pl.BlockSpec((blk, EDGE_DIM), lambda i: (i, 0)),
            wspec((HIDDEN, 145)), wspec((HIDDEN, 17)), wspec((EDGE_DIM, 17)),
            wspec((1, EDGE_DIM)),
            wspec((EDGE_DIM, EDGE_DIM)), wspec((1, EDGE_DIM)),
            wspec((1, 1)),
            wspec((1, EDGE_DIM)), wspec((1, EDGE_DIM)),
            wspec((EDGE_DIM, HIDDEN)), wspec((1, HIDDEN)),
            wspec((HIDDEN, HIDDEN)), wspec((1, HIDDEN)),
        ],
        out_specs=[
            pl.BlockSpec((blk, EDGE_DIM), lambda i: (i, 0)),
            pl.BlockSpec((blk, HIDDEN), lambda i: (i, 0)),
        ],
        out_shape=[
            jax.ShapeDtypeStruct((n_edges, EDGE_DIM), jnp.float32),
            jax.ShapeDtypeStruct((n_edges, HIDDEN), jnp.float32),
        ],
    )(gathered, gathered, edge_attr, Ws, Wd, Wa, be1, We2, be2,
      bg, ge, be, Wm1b, bm1, Wm2, bm2)


def _tc_node(h, parts, Wu1a, Wu1b, bu1, Wu2, bu2, gn, bn):
    n_nodes = h.shape[0]
    blk = 1000
    nb = n_nodes // blk

    def dot(a, b):
        return lax.dot_general(a, b, (((1,), (0,)), ((), ())),
                               preferred_element_type=jnp.float32)

    bf = jnp.bfloat16

    def body(h_r, p0_r, p1_r, Wu1a_r, Wu1b_r, bu1_r, Wu2_r, bu2_r, gn_r, bn_r,
             o_ref):
        hh = h_r[...]
        agg = p0_r[...] + p1_r[...]
        u = _gelu(dot(hh.astype(bf), Wu1a_r[...])
                  + dot(agg.astype(bf), Wu1b_r[...]) + bu1_r[...])
        h2 = dot(u.astype(bf), Wu2_r[...]) + bu2_r[...]
        x = hh + h2
        mu = jnp.mean(x, axis=-1, keepdims=True)
        var = jnp.mean((x - mu) ** 2, axis=-1, keepdims=True)
        o_ref[...] = (x - mu) / jnp.sqrt(var + 1e-5) * gn_r[...] + bn_r[...]

    wspec = lambda shp: pl.BlockSpec(shp, lambda i: (0, 0))
    return pl.pallas_call(
        body,
        grid=(nb,),
        in_specs=[
            pl.BlockSpec((blk, HIDDEN), lambda i: (i, 0)),
            pl.BlockSpec((blk, HIDDEN), lambda i: (i, 0)),
            pl.BlockSpec((blk, HIDDEN), lambda i: (i + nb, 0)),
            wspec((HIDDEN, HIDDEN)), wspec((HIDDEN, HIDDEN)),
            wspec((1, HIDDEN)),
            wspec((HIDDEN, HIDDEN)), wspec((1, HIDDEN)),
            wspec((1, HIDDEN)), wspec((1, HIDDEN)),
        ],
        out_specs=pl.BlockSpec((blk, HIDDEN), lambda i: (i, 0)),
        out_shape=jax.ShapeDtypeStruct((n_nodes, HIDDEN), jnp.float32),
    )(h, parts, parts, Wu1a, Wu1b, bu1, Wu2, bu2, gn, bn)


def kernel(h, edge_index, edge_attr, Wm1, bm1, Wm2, bm2, Wu1, bu1, Wu2, bu2,
           gn, bn, We1, be1, We2, be2, Wg, bg, ge, be):
    n_nodes = h.shape[0]
    H, D = HIDDEN, EDGE_DIM

    idx_flat = edge_index.reshape(-1)
    dst = edge_index[1]

    row = lambda v: v.reshape(1, -1)
    bf = jnp.bfloat16

    Ws = jnp.concatenate([Wm1[:H], We1[:H], Wg[:H]], axis=1).astype(bf)
    Wd = jnp.concatenate([We1[H:2 * H], Wg[H:2 * H]], axis=1).astype(bf)
    Wa = jnp.concatenate([We1[2 * H:], Wg[2 * H:]], axis=1).astype(bf)

    gathered = _sc_gather(h, idx_flat)
    e, m = _tc_edge(
        gathered, edge_attr, Ws, Wd, Wa, row(be1), We2.astype(bf), row(be2),
        row(bg), row(ge), row(be), Wm1[H:].astype(bf), row(bm1),
        Wm2.astype(bf), row(bm2))
    parts = _sc_scatter(m, dst, n_nodes)
    hn = _tc_node(h, parts, Wu1[:H].astype(bf), Wu1[H:].astype(bf), row(bu1),
                  Wu2.astype(bf), row(bu2), row(gn), row(bn))
    return (hn, e)


# speedup vs baseline: 3.2795x; 1.0004x over previous
import functools

import jax
import jax.numpy as jnp
from jax import lax
from jax.experimental import pallas as pl
from jax.experimental.pallas import tpu as pltpu
from jax.experimental.pallas import tpu_sc as plsc

HIDDEN = 128
EDGE_DIM = 16
EDGE_SCALE = 0.1
_NW = 32
_SQRT1_2 = 0.7071067811865476


def _gelu(x):
    return 0.5 * x * (1.0 + lax.erf(x * _SQRT1_2))


def _sc_gather(h, idx_flat):
    n_idx = idx_flat.shape[0]
    per_w = n_idx // _NW
    ch = 80
    n_ch = per_w // ch
    n_pair = n_ch // 2
    mesh = plsc.VectorSubcoreMesh(core_axis_name="c", subcore_axis_name="s")

    @functools.partial(
        pl.kernel,
        out_type=jax.ShapeDtypeStruct((n_idx, HIDDEN), jnp.float32),
        mesh=mesh,
        scratch_types=[
            pltpu.VMEM((per_w,), jnp.int32),
            pltpu.VMEM((ch, HIDDEN), jnp.float32),
            pltpu.VMEM((ch, HIDDEN), jnp.float32),
            pltpu.SemaphoreType.DMA,
            pltpu.SemaphoreType.DMA,
            pltpu.SemaphoreType.DMA,
            pltpu.SemaphoreType.DMA,
        ],
    )
    def k(h_hbm, idx_hbm, out_hbm, idx_all, rows0, rows1, gs0, gs1, ws0, ws1):
        c = lax.axis_index("c")
        s = lax.axis_index("s")
        base = (c * 16 + s) * per_w
        pltpu.sync_copy(idx_hbm.at[pl.ds(base, per_w)], idx_all)

        def gat(g, rows, sem):
            pltpu.async_copy(h_hbm.at[idx_all.at[pl.ds(g * ch, ch)]],
                             rows, sem)

        def wr(g, rows, sem):
            pltpu.async_copy(rows, out_hbm.at[pl.ds(base + g * ch, ch)], sem)

        def wr_wait(g, rows, sem):
            pltpu.make_async_copy(
                rows, out_hbm.at[pl.ds(base + g * ch, ch)], sem).wait()

        def g_wait(g, rows, sem):
            pltpu.make_async_copy(h_hbm.at[idx_all.at[pl.ds(g * ch, ch)]],
                                  rows, sem).wait()

        gat(0, rows0, gs0)

        def body(p, carry):
            c0 = 2 * p
            c1 = c0 + 1

            @pl.when(p > 0)
            def _():
                wr_wait(c1 - 2, rows1, ws1)

            gat(c1, rows1, gs1)
            g_wait(c0, rows0, gs0)
            wr(c0, rows0, ws0)
            g_wait(c1, rows1, gs1)
            wr(c1, rows1, ws1)

            @pl.when(p < n_pair - 1)
            def _():
                wr_wait(c0, rows0, ws0)
                gat(c0 + 2, rows0, gs0)

            return carry

        lax.fori_loop(0, n_pair, body, 0)
        wr_wait(n_ch - 2, rows0, ws0)
        wr_wait(n_ch - 1, rows1, ws1)

    return k(h, idx_flat)


def _sc_scatter(m, dst, n_nodes):
    n_edges = m.shape[0]
    per_w = n_edges // _NW
    ch = 80
    n_ch = per_w // ch
    n_rch = n_nodes // ch
    mesh = plsc.VectorSubcoreMesh(core_axis_name="c", subcore_axis_name="s")

    n_pair = n_ch // 2

    @functools.partial(
        pl.kernel,
        out_type=jax.ShapeDtypeStruct((2 * n_nodes, HIDDEN), jnp.float32),
        mesh=mesh,
        scratch_types=[
            pltpu.VMEM((ch,), jnp.int32),
            pltpu.VMEM((ch,), jnp.int32),
            pltpu.VMEM((ch, HIDDEN), jnp.float32),
            pltpu.VMEM((ch, HIDDEN), jnp.float32),
            pltpu.VMEM_SHARED((n_nodes, HIDDEN), jnp.float32),
            pltpu.SemaphoreType.DMA,
            pltpu.SemaphoreType.DMA,
            pltpu.SemaphoreType.DMA,
            pltpu.SemaphoreType.DMA,
        ],
    )
    def k(m_hbm, dst_hbm, out_hbm, i0, i1, m0, m1, acc_sh, ls0, ls1, ss0, ss1):
        c = lax.axis_index("c")
        s = lax.axis_index("s")

        zero16 = jnp.zeros((16,), jnp.float32)

        def zrow(i, carry):
            for j in range(HIDDEN // 16):
                m0[i, pl.ds(j * 16, 16)] = zero16
            return carry

        lax.fori_loop(0, ch, zrow, 0)

        n_mine = (n_rch - s + 15) // 16

        def zchunk(k_, carry):
            cid = s + k_ * 16
            pltpu.sync_copy(m0, acc_sh.at[pl.ds(cid * ch, ch)])
            return carry

        lax.fori_loop(0, n_mine, zchunk, 0)

        base = (c * 16 + s) * per_w
        plsc.subcore_barrier()

        def ld(g, buf, sem):
            pltpu.async_copy(m_hbm.at[pl.ds(base + g * ch, ch)], buf, sem)

        def ld_wait(g, buf, sem):
            pltpu.make_async_copy(m_hbm.at[pl.ds(base + g * ch, ch)], buf,
                                  sem).wait()

        def ldi(g, ibuf):
            pltpu.sync_copy(dst_hbm.at[pl.ds(base + g * ch, ch)], ibuf)

        def sc(ibuf, buf, sem):
            pltpu.async_copy(buf, acc_sh.at[ibuf], sem, add=True)

        def sc_wait(ibuf, buf, sem):
            pltpu.make_async_copy(buf, acc_sh.at[ibuf], sem).wait()

        ldi(0, i0)
        ld(0, m0, ls0)

        def body(p, carry):
            c0 = 2 * p
            c1 = c0 + 1
            ld(c1, m1, ls1)
            ldi(c1, i1)
            ld_wait(c0, m0, ls0)
            pltpu.sync_copy(m0, acc_sh.at[i0], add=True)

            @pl.when(p < n_pair - 1)
            def _():
                ld(c0 + 2, m0, ls0)
                ldi(c0 + 2, i0)

            ld_wait(c1, m1, ls1)
            pltpu.sync_copy(m1, acc_sh.at[i1], add=True)
            return carry

        lax.fori_loop(0, n_pair, body, 0)
        for g_tail in range(2 * n_pair, n_ch):
            ldi(g_tail, i0)
            pltpu.sync_copy(m_hbm.at[pl.ds(base + g_tail * ch, ch)], m0)
            pltpu.sync_copy(m0, acc_sh.at[i0], add=True)
        plsc.subcore_barrier()

        def wchunk(k_, carry):
            r = (s + k_ * 16) * ch
            pltpu.sync_copy(acc_sh.at[pl.ds(r, ch)], m0)
            pltpu.sync_copy(m0, out_hbm.at[pl.ds(c * n_nodes + r, ch)])
            return carry

        lax.fori_loop(0, n_mine, wchunk, 0)

    return k(m, dst)


def _tc_edge(gathered, edge_attr, Ws, Wd, Wa, be1, We2, be2, bg, ge, be,
             Wm1b, bm1, Wm2, bm2):
    n_edges = edge_attr.shape[0]
    blk = 1280
    nb = n_edges // blk
    bf = jnp.bfloat16

    def dot(a, b):
        return lax.dot_general(a, b, (((1,), (0,)), ((), ())),
                               preferred_element_type=jnp.float32)

    def body(hs_r, hd_r, ea_r, Ws_r, Wd_r, Wa_r, be1_r, We2_r, be2_r,
             bg_r, ge_r, be_r, Wm1b_r, bm1_r, Wm2_r, bm2_r, e_ref, m_ref):
        hs = hs_r[...].astype(bf)
        hd = hd_r[...].astype(bf)
        ea = ea_r[...]
        y_s = dot(hs, Ws_r[...])
        y_d = dot(hd, Wd_r[...])
        y_a = dot(ea.astype(bf), Wa_r[...])
        t = _gelu(y_s[:, HIDDEN:HIDDEN + EDGE_DIM]
                  + y_d[:, :EDGE_DIM] + y_a[:, :EDGE_DIM] + be1_r[...])
        g = (y_s[:, HIDDEN + EDGE_DIM:] + y_d[:, EDGE_DIM:]
             + y_a[:, EDGE_DIM:] + bg_r[...])
        gate = jax.nn.sigmoid(g)
        delta = (dot(t.astype(bf), We2_r[...]) + be2_r[...]) * gate
        x = ea + EDGE_SCALE * delta
        mu = jnp.mean(x, axis=-1, keepdims=True)
        var = jnp.mean((x - mu) ** 2, axis=-1, keepdims=True)
        e = (x - mu) / jnp.sqrt(var + 1e-5) * ge_r[...] + be_r[...]
        e_ref[...] = e
        u = _gelu(y_s[:, :HIDDEN] + dot(e.astype(bf), Wm1b_r[...]) + bm1_r[...])
        m_ref[...] = dot(u.astype(bf), Wm2_r[...]) + bm2_r[...]

    wspec = lambda shp: pl.BlockSpec(shp, lambda i: (0, 0))
    return pl.pallas_call(
        body,
        grid=(nb,),
        in_specs=[
            pl.BlockSpec((blk, HIDDEN), lambda i: (i, 0)),
            pl.BlockSpec((blk, HIDDEN), lambda i: (i + nb, 0)),
            pl.BlockSpec((blk, EDGE_DIM), lambda i: (i, 0)),
            wspec((HIDDEN, 145)), wspec((HIDDEN, 17)), wspec((EDGE_DIM, 17)),
            wspec((1, EDGE_DIM)),
            wspec((EDGE_DIM, EDGE_DIM)), wspec((1, EDGE_DIM)),
            wspec((1, 1)),
            wspec((1, EDGE_DIM)), wspec((1, EDGE_DIM)),
            wspec((EDGE_DIM, HIDDEN)), wspec((1, HIDDEN)),
            wspec((HIDDEN, HIDDEN)), wspec((1, HIDDEN)),
        ],
        out_specs=[
            pl.BlockSpec((blk, EDGE_DIM), lambda i: (i, 0)),
            pl.BlockSpec((blk, HIDDEN), lambda i: (i, 0)),
        ],
        out_shape=[
            jax.ShapeDtypeStruct((n_edges, EDGE_DIM), jnp.float32),
            jax.ShapeDtypeStruct((n_edges, HIDDEN), jnp.float32),
        ],
    )(gathered, gathered, edge_attr, Ws, Wd, Wa, be1, We2, be2,
      bg, ge, be, Wm1b, bm1, Wm2, bm2)


def _tc_node(h, parts, Wu1a, Wu1b, bu1, Wu2, bu2, gn, bn):
    n_nodes = h.shape[0]
    blk = 1000
    nb = n_nodes // blk

    def dot(a, b):
        return lax.dot_general(a, b, (((1,), (0,)), ((), ())),
                               preferred_element_type=jnp.float32)

    bf = jnp.bfloat16

    def body(h_r, p0_r, p1_r, Wu1a_r, Wu1b_r, bu1_r, Wu2_r, bu2_r, gn_r, bn_r,
             o_ref):
        hh = h_r[...]
        agg = p0_r[...] + p1_r[...]
        u = _gelu(dot(hh.astype(bf), Wu1a_r[...])
                  + dot(agg.astype(bf), Wu1b_r[...]) + bu1_r[...])
        h2 = dot(u.astype(bf), Wu2_r[...]) + bu2_r[...]
        x = hh + h2
        mu = jnp.mean(x, axis=-1, keepdims=True)
        var = jnp.mean((x - mu) ** 2, axis=-1, keepdims=True)
        o_ref[...] = (x - mu) / jnp.sqrt(var + 1e-5) * gn_r[...] + bn_r[...]

    wspec = lambda shp: pl.BlockSpec(shp, lambda i: (0, 0))
    return pl.pallas_call(
        body,
        grid=(nb,),
        in_specs=[
            pl.BlockSpec((blk, HIDDEN), lambda i: (i, 0)),
            pl.BlockSpec((blk, HIDDEN), lambda i: (i, 0)),
            pl.BlockSpec((blk, HIDDEN), lambda i: (i + nb, 0)),
            wspec((HIDDEN, HIDDEN)), wspec((HIDDEN, HIDDEN)),
            wspec((1, HIDDEN)),
            wspec((HIDDEN, HIDDEN)), wspec((1, HIDDEN)),
            wspec((1, HIDDEN)), wspec((1, HIDDEN)),
        ],
        out_specs=pl.BlockSpec((blk, HIDDEN), lambda i: (i, 0)),
        out_shape=jax.ShapeDtypeStruct((n_nodes, HIDDEN), jnp.float32),
    )(h, parts, parts, Wu1a, Wu1b, bu1, Wu2, bu2, gn, bn)


def kernel(h, edge_index, edge_attr, Wm1, bm1, Wm2, bm2, Wu1, bu1, Wu2, bu2,
           gn, bn, We1, be1, We2, be2, Wg, bg, ge, be):
    n_nodes = h.shape[0]
    H, D = HIDDEN, EDGE_DIM

    idx_flat = edge_index.reshape(-1)
    dst = edge_index[1]

    row = lambda v: v.reshape(1, -1)
    bf = jnp.bfloat16

    Ws = jnp.concatenate([Wm1[:H], We1[:H], Wg[:H]], axis=1).astype(bf)
    Wd = jnp.concatenate([We1[H:2 * H], Wg[H:2 * H]], axis=1).astype(bf)
    Wa = jnp.concatenate([We1[2 * H:], Wg[2 * H:]], axis=1).astype(bf)

    gathered = _sc_gather(h, idx_flat)
    e, m = _tc_edge(
        gathered, edge_attr, Ws, Wd, Wa, row(be1), We2.astype(bf), row(be2),
        row(bg), row(ge), row(be), Wm1[H:].astype(bf), row(bm1),
        Wm2.astype(bf), row(bm2))
    parts = _sc_scatter(m, dst, n_nodes)
    hn = _tc_node(h, parts, Wu1[:H].astype(bf), Wu1[H:].astype(bf), row(bu1),
                  Wu2.astype(bf), row(bu2), row(gn), row(bn))
    return (hn, e)


# speedup vs baseline: 3.5562x; 1.0844x over previous
import functools

import jax
import jax.numpy as jnp
from jax import lax
from jax.experimental import pallas as pl
from jax.experimental.pallas import tpu as pltpu
from jax.experimental.pallas import tpu_sc as plsc

HIDDEN = 128
EDGE_DIM = 16
EDGE_SCALE = 0.1
_NW = 32
_SQRT1_2 = 0.7071067811865476


def _gelu(x):
    return 0.5 * x * (1.0 + lax.erf(x * _SQRT1_2))


def _sc_gather(h, idx_flat):
    n_idx = idx_flat.shape[0]
    per_w = n_idx // _NW
    ch = 80
    n_ch = per_w // ch
    n_pair = n_ch // 2
    mesh = plsc.VectorSubcoreMesh(core_axis_name="c", subcore_axis_name="s")

    @functools.partial(
        pl.kernel,
        out_type=jax.ShapeDtypeStruct((n_idx, HIDDEN), jnp.float32),
        mesh=mesh,
        scratch_types=[
            pltpu.VMEM((per_w,), jnp.int32),
            pltpu.VMEM((ch, HIDDEN), jnp.float32),
            pltpu.VMEM((ch, HIDDEN), jnp.float32),
            pltpu.SemaphoreType.DMA,
            pltpu.SemaphoreType.DMA,
            pltpu.SemaphoreType.DMA,
            pltpu.SemaphoreType.DMA,
        ],
    )
    def k(h_hbm, idx_hbm, out_hbm, idx_all, rows0, rows1, gs0, gs1, ws0, ws1):
        c = lax.axis_index("c")
        s = lax.axis_index("s")
        base = (c * 16 + s) * per_w
        pltpu.sync_copy(idx_hbm.at[pl.ds(base, per_w)], idx_all)

        def gat(g, rows, sem):
            pltpu.async_copy(h_hbm.at[idx_all.at[pl.ds(g * ch, ch)]],
                             rows, sem)

        def wr(g, rows, sem):
            pltpu.async_copy(rows, out_hbm.at[pl.ds(base + g * ch, ch)], sem)

        def wr_wait(g, rows, sem):
            pltpu.make_async_copy(
                rows, out_hbm.at[pl.ds(base + g * ch, ch)], sem).wait()

        def g_wait(g, rows, sem):
            pltpu.make_async_copy(h_hbm.at[idx_all.at[pl.ds(g * ch, ch)]],
                                  rows, sem).wait()

        gat(0, rows0, gs0)

        def body(p, carry):
            c0 = 2 * p
            c1 = c0 + 1

            @pl.when(p > 0)
            def _():
                wr_wait(c1 - 2, rows1, ws1)

            gat(c1, rows1, gs1)
            g_wait(c0, rows0, gs0)
            wr(c0, rows0, ws0)
            g_wait(c1, rows1, gs1)
            wr(c1, rows1, ws1)

            @pl.when(p < n_pair - 1)
            def _():
                wr_wait(c0, rows0, ws0)
                gat(c0 + 2, rows0, gs0)

            return carry

        lax.fori_loop(0, n_pair, body, 0)
        wr_wait(2 * n_pair - 2, rows0, ws0)
        wr_wait(2 * n_pair - 1, rows1, ws1)
        for g_tail in range(2 * n_pair, n_ch):
            gat(g_tail, rows0, gs0)
            g_wait(g_tail, rows0, gs0)
            pltpu.sync_copy(rows0, out_hbm.at[pl.ds(base + g_tail * ch, ch)])

    return k(h, idx_flat)


def _sc_scatter(m, dst, n_nodes):
    n_edges = m.shape[0]
    per_w = n_edges // _NW
    ch = 80 if per_w % 80 == 0 else 40
    n_ch = per_w // ch
    n_rch = n_nodes // ch
    mesh = plsc.VectorSubcoreMesh(core_axis_name="c", subcore_axis_name="s")

    n_pair = n_ch // 2

    @functools.partial(
        pl.kernel,
        out_type=jax.ShapeDtypeStruct((2 * n_nodes, HIDDEN), jnp.float32),
        mesh=mesh,
        scratch_types=[
            pltpu.VMEM((ch,), jnp.int32),
            pltpu.VMEM((ch,), jnp.int32),
            pltpu.VMEM((ch, HIDDEN), jnp.float32),
            pltpu.VMEM((ch, HIDDEN), jnp.float32),
            pltpu.VMEM_SHARED((n_nodes, HIDDEN), jnp.float32),
            pltpu.SemaphoreType.DMA,
            pltpu.SemaphoreType.DMA,
            pltpu.SemaphoreType.DMA,
            pltpu.SemaphoreType.DMA,
        ],
    )
    def k(m_hbm, dst_hbm, out_hbm, i0, i1, m0, m1, acc_sh, ls0, ls1, ss0, ss1):
        c = lax.axis_index("c")
        s = lax.axis_index("s")

        zero16 = jnp.zeros((16,), jnp.float32)

        def zrow(i, carry):
            for j in range(HIDDEN // 16):
                m0[i, pl.ds(j * 16, 16)] = zero16
            return carry

        lax.fori_loop(0, ch, zrow, 0)

        n_mine = (n_rch - s + 15) // 16

        def zchunk(k_, carry):
            cid = s + k_ * 16
            pltpu.sync_copy(m0, acc_sh.at[pl.ds(cid * ch, ch)])
            return carry

        lax.fori_loop(0, n_mine, zchunk, 0)

        base = (c * 16 + s) * per_w
        plsc.subcore_barrier()

        def ld(g, buf, sem):
            pltpu.async_copy(m_hbm.at[pl.ds(base + g * ch, ch)], buf, sem)

        def ld_wait(g, buf, sem):
            pltpu.make_async_copy(m_hbm.at[pl.ds(base + g * ch, ch)], buf,
                                  sem).wait()

        def ldi(g, ibuf):
            pltpu.sync_copy(dst_hbm.at[pl.ds(base + g * ch, ch)], ibuf)

        def sc(ibuf, buf, sem):
            pltpu.async_copy(buf, acc_sh.at[ibuf], sem, add=True)

        def sc_wait(ibuf, buf, sem):
            pltpu.make_async_copy(buf, acc_sh.at[ibuf], sem).wait()

        ldi(0, i0)
        ld(0, m0, ls0)

        def body(p, carry):
            c0 = 2 * p
            c1 = c0 + 1
            ld(c1, m1, ls1)
            ldi(c1, i1)
            ld_wait(c0, m0, ls0)
            pltpu.sync_copy(m0, acc_sh.at[i0], add=True)

            @pl.when(p < n_pair - 1)
            def _():
                ld(c0 + 2, m0, ls0)
                ldi(c0 + 2, i0)

            ld_wait(c1, m1, ls1)
            pltpu.sync_copy(m1, acc_sh.at[i1], add=True)
            return carry

        lax.fori_loop(0, n_pair, body, 0)
        for g_tail in range(2 * n_pair, n_ch):
            ldi(g_tail, i0)
            pltpu.sync_copy(m_hbm.at[pl.ds(base + g_tail * ch, ch)], m0)
            pltpu.sync_copy(m0, acc_sh.at[i0], add=True)
        plsc.subcore_barrier()

        def wchunk(k_, carry):
            r = (s + k_ * 16) * ch
            pltpu.sync_copy(acc_sh.at[pl.ds(r, ch)], m0)
            pltpu.sync_copy(m0, out_hbm.at[pl.ds(c * n_nodes + r, ch)])
            return carry

        lax.fori_loop(0, n_mine, wchunk, 0)

    return k(m, dst)


def _tc_edge(gathered, edge_attr, Ws, Wd, Wa, be1, We2, be2, bg, ge, be,
             Wm1b, bm1, Wm2, bm2, eoff):
    n_edges = gathered.shape[0] // 2
    blk = 1280
    nb = n_edges // blk
    bf = jnp.bfloat16

    def dot(a, b):
        return lax.dot_general(a, b, (((1,), (0,)), ((), ())),
                               preferred_element_type=jnp.float32)

    def body(hs_r, hd_r, ea_r, Ws_r, Wd_r, Wa_r, be1_r, We2_r, be2_r,
             bg_r, ge_r, be_r, Wm1b_r, bm1_r, Wm2_r, bm2_r, e_ref, m_ref):
        hs = hs_r[...].astype(bf)
        hd = hd_r[...].astype(bf)
        ea = ea_r[...]
        y_s = dot(hs, Ws_r[...])
        y_d = dot(hd, Wd_r[...])
        y_a = dot(ea.astype(bf), Wa_r[...])
        t = _gelu(y_s[:, HIDDEN:HIDDEN + EDGE_DIM]
                  + y_d[:, :EDGE_DIM] + y_a[:, :EDGE_DIM] + be1_r[...])
        g = (y_s[:, HIDDEN + EDGE_DIM:] + y_d[:, EDGE_DIM:]
             + y_a[:, EDGE_DIM:] + bg_r[...])
        gate = jax.nn.sigmoid(g)
        delta = (dot(t.astype(bf), We2_r[...]) + be2_r[...]) * gate
        x = ea + EDGE_SCALE * delta
        mu = jnp.mean(x, axis=-1, keepdims=True)
        var = jnp.mean((x - mu) ** 2, axis=-1, keepdims=True)
        e = (x - mu) / jnp.sqrt(var + 1e-5) * ge_r[...] + be_r[...]
        e_ref[...] = e
        u = _gelu(y_s[:, :HIDDEN] + dot(e.astype(bf), Wm1b_r[...]) + bm1_r[...])
        m_ref[...] = dot(u.astype(bf), Wm2_r[...]) + bm2_r[...]

    wspec = lambda shp: pl.BlockSpec(shp, lambda i: (0, 0))
    return pl.pallas_call(
        body,
        grid=(nb,),
        in_specs=[
            pl.BlockSpec((blk, HIDDEN), lambda i: (i, 0)),
            pl.BlockSpec((blk, HIDDEN), lambda i: (i + nb, 0)),
            pl.BlockSpec((blk, EDGE_DIM), lambda i: (i + eoff, 0)),
            wspec((HIDDEN, 145)), wspec((HIDDEN, 17)), wspec((EDGE_DIM, 17)),
            wspec((1, EDGE_DIM)),
            wspec((EDGE_DIM, EDGE_DIM)), wspec((1, EDGE_DIM)),
            wspec((1, 1)),
            wspec((1, EDGE_DIM)), wspec((1, EDGE_DIM)),
            wspec((EDGE_DIM, HIDDEN)), wspec((1, HIDDEN)),
            wspec((HIDDEN, HIDDEN)), wspec((1, HIDDEN)),
        ],
        out_specs=[
            pl.BlockSpec((blk, EDGE_DIM), lambda i: (i, 0)),
            pl.BlockSpec((blk, HIDDEN), lambda i: (i, 0)),
        ],
        out_shape=[
            jax.ShapeDtypeStruct((n_edges, EDGE_DIM), jnp.float32),
            jax.ShapeDtypeStruct((n_edges, HIDDEN), jnp.float32),
        ],
    )(gathered, gathered, edge_attr, Ws, Wd, Wa, be1, We2, be2,
      bg, ge, be, Wm1b, bm1, Wm2, bm2)


def _tc_node(h, parts_list, Wu1a, Wu1b, bu1, Wu2, bu2, gn, bn):
    n_nodes = h.shape[0]
    blk = 1000
    nb = n_nodes // blk
    np_ = len(parts_list)

    def dot(a, b):
        return lax.dot_general(a, b, (((1,), (0,)), ((), ())),
                               preferred_element_type=jnp.float32)

    bf = jnp.bfloat16

    def body(h_r, *rest):
        p_rs = rest[:2 * np_]
        Wu1a_r, Wu1b_r, bu1_r, Wu2_r, bu2_r, gn_r, bn_r, o_ref = rest[2 * np_:]
        hh = h_r[...]
        agg = p_rs[0][...]
        for p_r in p_rs[1:]:
            agg = agg + p_r[...]
        u = _gelu(dot(hh.astype(bf), Wu1a_r[...])
                  + dot(agg.astype(bf), Wu1b_r[...]) + bu1_r[...])
        h2 = dot(u.astype(bf), Wu2_r[...]) + bu2_r[...]
        x = hh + h2
        mu = jnp.mean(x, axis=-1, keepdims=True)
        var = jnp.mean((x - mu) ** 2, axis=-1, keepdims=True)
        o_ref[...] = (x - mu) / jnp.sqrt(var + 1e-5) * gn_r[...] + bn_r[...]

    wspec = lambda shp: pl.BlockSpec(shp, lambda i: (0, 0))
    part_specs = []
    part_args = []
    for parts in parts_list:
        part_specs += [pl.BlockSpec((blk, HIDDEN), lambda i: (i, 0)),
                       pl.BlockSpec((blk, HIDDEN), lambda i: (i + nb, 0))]
        part_args += [parts, parts]
    return pl.pallas_call(
        body,
        grid=(nb,),
        in_specs=[pl.BlockSpec((blk, HIDDEN), lambda i: (i, 0))] + part_specs + [
            wspec((HIDDEN, HIDDEN)), wspec((HIDDEN, HIDDEN)),
            wspec((1, HIDDEN)),
            wspec((HIDDEN, HIDDEN)), wspec((1, HIDDEN)),
            wspec((1, HIDDEN)), wspec((1, HIDDEN)),
        ],
        out_specs=pl.BlockSpec((blk, HIDDEN), lambda i: (i, 0)),
        out_shape=jax.ShapeDtypeStruct((n_nodes, HIDDEN), jnp.float32),
    )(h, *part_args, Wu1a, Wu1b, bu1, Wu2, bu2, gn, bn)


def kernel(h, edge_index, edge_attr, Wm1, bm1, Wm2, bm2, Wu1, bu1, Wu2, bu2,
           gn, bn, We1, be1, We2, be2, Wg, bg, ge, be):
    n_nodes = h.shape[0]
    H, D = HIDDEN, EDGE_DIM

    n_edges = edge_attr.shape[0]

    row = lambda v: v.reshape(1, -1)
    bf = jnp.bfloat16

    Ws = jnp.concatenate([Wm1[:H], We1[:H], Wg[:H]], axis=1).astype(bf)
    Wd = jnp.concatenate([We1[H:2 * H], Wg[H:2 * H]], axis=1).astype(bf)
    Wa = jnp.concatenate([We1[2 * H:], Wg[2 * H:]], axis=1).astype(bf)

    ns = 2
    es = n_edges // ns
    e_list, parts_list = [], []
    for j in range(ns):
        src_j = lax.dynamic_slice_in_dim(edge_index[0], j * es, es)
        dst_j = lax.dynamic_slice_in_dim(edge_index[1], j * es, es)
        idx_j = jnp.concatenate([src_j, dst_j])
        g_j = _sc_gather(h, idx_j)
        e_j, m_j = _tc_edge(
            g_j, edge_attr, Ws, Wd, Wa, row(be1), We2.astype(bf), row(be2),
            row(bg), row(ge), row(be), Wm1[H:].astype(bf), row(bm1),
            Wm2.astype(bf), row(bm2), j * (es // 1280))
        parts_list.append(_sc_scatter(m_j, dst_j, n_nodes))
        e_list.append(e_j)
    e = jnp.concatenate(e_list, axis=0)
    hn = _tc_node(h, parts_list, Wu1[:H].astype(bf), Wu1[H:].astype(bf),
                  row(bu1), Wu2.astype(bf), row(bu2), row(gn), row(bn))
    return (hn, e)


# speedup vs baseline: 3.6627x; 1.0299x over previous
import functools

import jax
import jax.numpy as jnp
from jax import lax
from jax.experimental import pallas as pl
from jax.experimental.pallas import tpu as pltpu
from jax.experimental.pallas import tpu_sc as plsc

HIDDEN = 128
EDGE_DIM = 16
EDGE_SCALE = 0.1
_NW = 32
_SQRT1_2 = 0.7071067811865476


def _gelu(x):
    return 0.5 * x * (1.0 + lax.erf(x * _SQRT1_2))


def _sc_gather(h, idx_flat):
    n_idx = idx_flat.shape[0]
    per_w = n_idx // _NW
    ch = 80
    n_ch = per_w // ch
    n_pair = n_ch // 2
    mesh = plsc.VectorSubcoreMesh(core_axis_name="c", subcore_axis_name="s")

    @functools.partial(
        pl.kernel,
        out_type=jax.ShapeDtypeStruct((n_idx, HIDDEN), jnp.float32),
        mesh=mesh,
        scratch_types=[
            pltpu.VMEM((per_w,), jnp.int32),
            pltpu.VMEM((ch, HIDDEN), jnp.float32),
            pltpu.VMEM((ch, HIDDEN), jnp.float32),
            pltpu.SemaphoreType.DMA,
            pltpu.SemaphoreType.DMA,
            pltpu.SemaphoreType.DMA,
            pltpu.SemaphoreType.DMA,
        ],
    )
    def k(h_hbm, idx_hbm, out_hbm, idx_all, rows0, rows1, gs0, gs1, ws0, ws1):
        c = lax.axis_index("c")
        s = lax.axis_index("s")
        base = (c * 16 + s) * per_w
        pltpu.sync_copy(idx_hbm.at[pl.ds(base, per_w)], idx_all)

        def gat(g, rows, sem):
            pltpu.async_copy(h_hbm.at[idx_all.at[pl.ds(g * ch, ch)]],
                             rows, sem)

        def wr(g, rows, sem):
            pltpu.async_copy(rows, out_hbm.at[pl.ds(base + g * ch, ch)], sem)

        def wr_wait(g, rows, sem):
            pltpu.make_async_copy(
                rows, out_hbm.at[pl.ds(base + g * ch, ch)], sem).wait()

        def g_wait(g, rows, sem):
            pltpu.make_async_copy(h_hbm.at[idx_all.at[pl.ds(g * ch, ch)]],
                                  rows, sem).wait()

        gat(0, rows0, gs0)

        def body(p, carry):
            c0 = 2 * p
            c1 = c0 + 1

            @pl.when(p > 0)
            def _():
                wr_wait(c1 - 2, rows1, ws1)

            gat(c1, rows1, gs1)
            g_wait(c0, rows0, gs0)
            wr(c0, rows0, ws0)
            g_wait(c1, rows1, gs1)
            wr(c1, rows1, ws1)

            @pl.when(p < n_pair - 1)
            def _():
                wr_wait(c0, rows0, ws0)
                gat(c0 + 2, rows0, gs0)

            return carry

        lax.fori_loop(0, n_pair, body, 0)
        wr_wait(2 * n_pair - 2, rows0, ws0)
        wr_wait(2 * n_pair - 1, rows1, ws1)
        for g_tail in range(2 * n_pair, n_ch):
            gat(g_tail, rows0, gs0)
            g_wait(g_tail, rows0, gs0)
            pltpu.sync_copy(rows0, out_hbm.at[pl.ds(base + g_tail * ch, ch)])

    return k(h, idx_flat)


def _sc_scatter(m, dst, n_nodes):
    n_edges = m.shape[0]
    per_w = n_edges // _NW
    ch = 80 if per_w % 80 == 0 else 40
    n_ch = per_w // ch
    n_rch = n_nodes // ch
    mesh = plsc.VectorSubcoreMesh(core_axis_name="c", subcore_axis_name="s")

    n_pair = n_ch // 2

    @functools.partial(
        pl.kernel,
        out_type=jax.ShapeDtypeStruct((2 * n_nodes, HIDDEN), jnp.float32),
        mesh=mesh,
        scratch_types=[
            pltpu.VMEM((ch,), jnp.int32),
            pltpu.VMEM((ch,), jnp.int32),
            pltpu.VMEM((ch, HIDDEN), jnp.float32),
            pltpu.VMEM((ch, HIDDEN), jnp.float32),
            pltpu.VMEM_SHARED((n_nodes, HIDDEN), jnp.float32),
            pltpu.SemaphoreType.DMA,
            pltpu.SemaphoreType.DMA,
            pltpu.SemaphoreType.DMA,
            pltpu.SemaphoreType.DMA,
        ],
    )
    def k(m_hbm, dst_hbm, out_hbm, i0, i1, m0, m1, acc_sh, ls0, ls1, ss0, ss1):
        c = lax.axis_index("c")
        s = lax.axis_index("s")

        zero16 = jnp.zeros((16,), jnp.float32)

        def zrow(i, carry):
            for j in range(HIDDEN // 16):
                m0[i, pl.ds(j * 16, 16)] = zero16
            return carry

        lax.fori_loop(0, ch, zrow, 0)

        n_mine = (n_rch - s + 15) // 16

        def zchunk(k_, carry):
            cid = s + k_ * 16
            pltpu.sync_copy(m0, acc_sh.at[pl.ds(cid * ch, ch)])
            return carry

        lax.fori_loop(0, n_mine, zchunk, 0)

        base = (c * 16 + s) * per_w
        plsc.subcore_barrier()

        def ld(g, buf, sem):
            pltpu.async_copy(m_hbm.at[pl.ds(base + g * ch, ch)], buf, sem)

        def ld_wait(g, buf, sem):
            pltpu.make_async_copy(m_hbm.at[pl.ds(base + g * ch, ch)], buf,
                                  sem).wait()

        def ldi(g, ibuf):
            pltpu.sync_copy(dst_hbm.at[pl.ds(base + g * ch, ch)], ibuf)

        def sc(ibuf, buf, sem):
            pltpu.async_copy(buf, acc_sh.at[ibuf], sem, add=True)

        def sc_wait(ibuf, buf, sem):
            pltpu.make_async_copy(buf, acc_sh.at[ibuf], sem).wait()

        ldi(0, i0)
        ld(0, m0, ls0)

        def body(p, carry):
            c0 = 2 * p
            c1 = c0 + 1
            ld(c1, m1, ls1)
            ldi(c1, i1)
            ld_wait(c0, m0, ls0)
            pltpu.sync_copy(m0, acc_sh.at[i0], add=True)

            @pl.when(p < n_pair - 1)
            def _():
                ld(c0 + 2, m0, ls0)
                ldi(c0 + 2, i0)

            ld_wait(c1, m1, ls1)
            pltpu.sync_copy(m1, acc_sh.at[i1], add=True)
            return carry

        lax.fori_loop(0, n_pair, body, 0)
        for g_tail in range(2 * n_pair, n_ch):
            ldi(g_tail, i0)
            pltpu.sync_copy(m_hbm.at[pl.ds(base + g_tail * ch, ch)], m0)
            pltpu.sync_copy(m0, acc_sh.at[i0], add=True)
        plsc.subcore_barrier()

        def wchunk(k_, carry):
            r = (s + k_ * 16) * ch
            pltpu.sync_copy(acc_sh.at[pl.ds(r, ch)], m0)
            pltpu.sync_copy(m0, out_hbm.at[pl.ds(c * n_nodes + r, ch)])
            return carry

        lax.fori_loop(0, n_mine, wchunk, 0)

    return k(m, dst)


def _tc_edge(gathered, edge_attr, Ws, Wd, Wa, be1, We2, be2, bg, ge, be,
             Wm1b, bm1, Wm2, bm2, eoff):
    n_edges = gathered.shape[0] // 2
    blk = 1280
    nb = n_edges // blk
    bf = jnp.bfloat16

    def dot(a, b):
        return lax.dot_general(a, b, (((1,), (0,)), ((), ())),
                               preferred_element_type=jnp.float32)

    def body(hs_r, hd_r, ea_r, Ws_r, Wd_r, Wa_r, be1_r, We2_r, be2_r,
             bg_r, ge_r, be_r, Wm1b_r, bm1_r, Wm2_r, bm2_r, e_ref, m_ref):
        hs = hs_r[...].astype(bf)
        hd = hd_r[...].astype(bf)
        ea = ea_r[...]
        y_s = dot(hs, Ws_r[...])
        y_d = dot(hd, Wd_r[...])
        y_a = dot(ea.astype(bf), Wa_r[...])
        tg = y_s[:, HIDDEN:] + y_d + y_a
        t = _gelu(tg[:, :EDGE_DIM] + be1_r[...])
        gate = 0.5 * jnp.tanh((tg[:, EDGE_DIM:] + bg_r[...]) * 0.5) + 0.5
        x = ea + (dot(t.astype(bf), We2_r[...]) + be2_r[...]) * gate

        jmean = jnp.full((EDGE_DIM, EDGE_DIM), 1.0 / EDGE_DIM, bf)
        xc = x - dot(x.astype(bf), jmean)
        var = dot((xc * xc).astype(bf), jmean)
        e = xc * lax.rsqrt(var + 1e-5) * ge_r[...] + be_r[...]
        e_ref[...] = e
        u = _gelu(y_s[:, :HIDDEN] + dot(e.astype(bf), Wm1b_r[...]) + bm1_r[...])
        m_ref[...] = dot(u.astype(bf), Wm2_r[...]) + bm2_r[...]

    wspec = lambda shp: pl.BlockSpec(shp, lambda i: (0, 0))
    return pl.pallas_call(
        body,
        grid=(nb,),
        in_specs=[
            pl.BlockSpec((blk, HIDDEN), lambda i: (i, 0)),
            pl.BlockSpec((blk, HIDDEN), lambda i: (i + nb, 0)),
            pl.BlockSpec((blk, EDGE_DIM), lambda i: (i + eoff, 0)),
            wspec((HIDDEN, 145)), wspec((HIDDEN, 17)), wspec((EDGE_DIM, 17)),
            wspec((1, EDGE_DIM)),
            wspec((EDGE_DIM, EDGE_DIM)), wspec((1, EDGE_DIM)),
            wspec((1, 1)),
            wspec((1, EDGE_DIM)), wspec((1, EDGE_DIM)),
            wspec((EDGE_DIM, HIDDEN)), wspec((1, HIDDEN)),
            wspec((HIDDEN, HIDDEN)), wspec((1, HIDDEN)),
        ],
        out_specs=[
            pl.BlockSpec((blk, EDGE_DIM), lambda i: (i, 0)),
            pl.BlockSpec((blk, HIDDEN), lambda i: (i, 0)),
        ],
        out_shape=[
            jax.ShapeDtypeStruct((n_edges, EDGE_DIM), jnp.float32),
            jax.ShapeDtypeStruct((n_edges, HIDDEN), jnp.float32),
        ],
    )(gathered, gathered, edge_attr, Ws, Wd, Wa, be1, We2, be2,
      bg, ge, be, Wm1b, bm1, Wm2, bm2)


def _tc_node(h, parts_list, Wu1a, Wu1b, bu1, Wu2, bu2, gn, bn):
    n_nodes = h.shape[0]
    blk = 1000
    nb = n_nodes // blk
    np_ = len(parts_list)

    def dot(a, b):
        return lax.dot_general(a, b, (((1,), (0,)), ((), ())),
                               preferred_element_type=jnp.float32)

    bf = jnp.bfloat16

    def body(h_r, *rest):
        p_rs = rest[:2 * np_]
        Wu1a_r, Wu1b_r, bu1_r, Wu2_r, bu2_r, gn_r, bn_r, o_ref = rest[2 * np_:]
        hh = h_r[...]
        agg = p_rs[0][...]
        for p_r in p_rs[1:]:
            agg = agg + p_r[...]
        u = _gelu(dot(hh.astype(bf), Wu1a_r[...])
                  + dot(agg.astype(bf), Wu1b_r[...]) + bu1_r[...])
        h2 = dot(u.astype(bf), Wu2_r[...]) + bu2_r[...]
        x = hh + h2
        mu = jnp.mean(x, axis=-1, keepdims=True)
        var = jnp.mean((x - mu) ** 2, axis=-1, keepdims=True)
        o_ref[...] = (x - mu) / jnp.sqrt(var + 1e-5) * gn_r[...] + bn_r[...]

    wspec = lambda shp: pl.BlockSpec(shp, lambda i: (0, 0))
    part_specs = []
    part_args = []
    for parts in parts_list:
        part_specs += [pl.BlockSpec((blk, HIDDEN), lambda i: (i, 0)),
                       pl.BlockSpec((blk, HIDDEN), lambda i: (i + nb, 0))]
        part_args += [parts, parts]
    return pl.pallas_call(
        body,
        grid=(nb,),
        in_specs=[pl.BlockSpec((blk, HIDDEN), lambda i: (i, 0))] + part_specs + [
            wspec((HIDDEN, HIDDEN)), wspec((HIDDEN, HIDDEN)),
            wspec((1, HIDDEN)),
            wspec((HIDDEN, HIDDEN)), wspec((1, HIDDEN)),
            wspec((1, HIDDEN)), wspec((1, HIDDEN)),
        ],
        out_specs=pl.BlockSpec((blk, HIDDEN), lambda i: (i, 0)),
        out_shape=jax.ShapeDtypeStruct((n_nodes, HIDDEN), jnp.float32),
    )(h, *part_args, Wu1a, Wu1b, bu1, Wu2, bu2, gn, bn)


def kernel(h, edge_index, edge_attr, Wm1, bm1, Wm2, bm2, Wu1, bu1, Wu2, bu2,
           gn, bn, We1, be1, We2, be2, Wg, bg, ge, be):
    n_nodes = h.shape[0]
    H, D = HIDDEN, EDGE_DIM

    n_edges = edge_attr.shape[0]

    row = lambda v: v.reshape(1, -1)
    bf = jnp.bfloat16

    Ws = jnp.concatenate([Wm1[:H], We1[:H], Wg[:H]], axis=1).astype(bf)
    Wd = jnp.concatenate([We1[H:2 * H], Wg[H:2 * H]], axis=1).astype(bf)
    Wa = jnp.concatenate([We1[2 * H:], Wg[2 * H:]], axis=1).astype(bf)

    ns = 2
    es = n_edges // ns
    e_list, parts_list = [], []
    for j in range(ns):
        src_j = lax.dynamic_slice_in_dim(edge_index[0], j * es, es)
        dst_j = lax.dynamic_slice_in_dim(edge_index[1], j * es, es)
        idx_j = jnp.concatenate([src_j, dst_j])
        g_j = _sc_gather(h, idx_j)
        e_j, m_j = _tc_edge(
            g_j, edge_attr, Ws, Wd, Wa, row(be1),
            (EDGE_SCALE * We2).astype(bf), row(EDGE_SCALE * be2),
            row(bg), row(ge), row(be), Wm1[H:].astype(bf), row(bm1),
            Wm2.astype(bf), row(bm2), j * (es // 1280))
        parts_list.append(_sc_scatter(m_j, dst_j, n_nodes))
        e_list.append(e_j)
    e = jnp.concatenate(e_list, axis=0)
    hn = _tc_node(h, parts_list, Wu1[:H].astype(bf), Wu1[H:].astype(bf),
                  row(bu1), Wu2.astype(bf), row(bu2), row(gn), row(bn))
    return (hn, e)


# speedup vs baseline: 3.7269x; 1.0175x over previous
import functools

import jax
import jax.numpy as jnp
from jax import lax
from jax.experimental import pallas as pl
from jax.experimental.pallas import tpu as pltpu
from jax.experimental.pallas import tpu_sc as plsc

HIDDEN = 128
EDGE_DIM = 16
EDGE_SCALE = 0.1
_NW = 32
_SQRT1_2 = 0.7071067811865476


def _gelu(x):
    return 0.5 * x * (1.0 + lax.erf(x * _SQRT1_2))


def _sc_gather(h, idx_flat):
    n_idx = idx_flat.shape[0]
    per_w = n_idx // _NW
    ch = 80
    n_ch = per_w // ch
    n_pair = n_ch // 2
    mesh = plsc.VectorSubcoreMesh(core_axis_name="c", subcore_axis_name="s")

    @functools.partial(
        pl.kernel,
        out_type=jax.ShapeDtypeStruct((n_idx, HIDDEN), jnp.float32),
        mesh=mesh,
        scratch_types=[
            pltpu.VMEM((per_w,), jnp.int32),
            pltpu.VMEM((ch, HIDDEN), jnp.float32),
            pltpu.VMEM((ch, HIDDEN), jnp.float32),
            pltpu.SemaphoreType.DMA,
            pltpu.SemaphoreType.DMA,
            pltpu.SemaphoreType.DMA,
            pltpu.SemaphoreType.DMA,
        ],
    )
    def k(h_hbm, idx_hbm, out_hbm, idx_all, rows0, rows1, gs0, gs1, ws0, ws1):
        c = lax.axis_index("c")
        s = lax.axis_index("s")
        base = (c * 16 + s) * per_w
        pltpu.sync_copy(idx_hbm.at[pl.ds(base, per_w)], idx_all)

        def gat(g, rows, sem):
            pltpu.async_copy(h_hbm.at[idx_all.at[pl.ds(g * ch, ch)]],
                             rows, sem)

        def wr(g, rows, sem):
            pltpu.async_copy(rows, out_hbm.at[pl.ds(base + g * ch, ch)], sem)

        def wr_wait(g, rows, sem):
            pltpu.make_async_copy(
                rows, out_hbm.at[pl.ds(base + g * ch, ch)], sem).wait()

        def g_wait(g, rows, sem):
            pltpu.make_async_copy(h_hbm.at[idx_all.at[pl.ds(g * ch, ch)]],
                                  rows, sem).wait()

        gat(0, rows0, gs0)

        def body(p, carry):
            c0 = 2 * p
            c1 = c0 + 1

            @pl.when(p > 0)
            def _():
                wr_wait(c1 - 2, rows1, ws1)

            gat(c1, rows1, gs1)
            g_wait(c0, rows0, gs0)
            wr(c0, rows0, ws0)
            g_wait(c1, rows1, gs1)
            wr(c1, rows1, ws1)

            @pl.when(p < n_pair - 1)
            def _():
                wr_wait(c0, rows0, ws0)
                gat(c0 + 2, rows0, gs0)

            return carry

        lax.fori_loop(0, n_pair, body, 0)
        wr_wait(2 * n_pair - 2, rows0, ws0)
        wr_wait(2 * n_pair - 1, rows1, ws1)
        for g_tail in range(2 * n_pair, n_ch):
            gat(g_tail, rows0, gs0)
            g_wait(g_tail, rows0, gs0)
            pltpu.sync_copy(rows0, out_hbm.at[pl.ds(base + g_tail * ch, ch)])

    return k(h, idx_flat)


def _sc_scatter(m, dst, n_nodes):
    n_edges = m.shape[0]
    ch = 80
    n_chunks = n_edges // ch
    nbase = n_chunks // _NW
    extra = n_chunks % _NW
    n_rch = n_nodes // ch
    mesh = plsc.VectorSubcoreMesh(core_axis_name="c", subcore_axis_name="s")

    n_pair = nbase // 2

    @functools.partial(
        pl.kernel,
        out_type=jax.ShapeDtypeStruct((2 * n_nodes, HIDDEN), jnp.float32),
        mesh=mesh,
        scratch_types=[
            pltpu.VMEM((ch,), jnp.int32),
            pltpu.VMEM((ch,), jnp.int32),
            pltpu.VMEM((ch, HIDDEN), jnp.float32),
            pltpu.VMEM((ch, HIDDEN), jnp.float32),
            pltpu.VMEM_SHARED((n_nodes, HIDDEN), jnp.float32),
            pltpu.SemaphoreType.DMA,
            pltpu.SemaphoreType.DMA,
            pltpu.SemaphoreType.DMA,
            pltpu.SemaphoreType.DMA,
        ],
    )
    def k(m_hbm, dst_hbm, out_hbm, i0, i1, m0, m1, acc_sh, ls0, ls1, ss0, ss1):
        c = lax.axis_index("c")
        s = lax.axis_index("s")

        zero16 = jnp.zeros((16,), jnp.float32)

        def zrow(i, carry):
            for j in range(HIDDEN // 16):
                m0[i, pl.ds(j * 16, 16)] = zero16
            return carry

        lax.fori_loop(0, ch, zrow, 0)

        n_mine = (n_rch - s + 15) // 16

        def zchunk(k_, carry):
            cid = s + k_ * 16
            pltpu.sync_copy(m0, acc_sh.at[pl.ds(cid * ch, ch)])
            return carry

        lax.fori_loop(0, n_mine, zchunk, 0)

        wid = c * 16 + s
        n_mine_e = nbase + jnp.where(wid < extra, 1, 0)
        base = ch * (nbase * wid + jnp.minimum(wid, extra))
        plsc.subcore_barrier()

        def ld(g, buf, sem):
            pltpu.async_copy(m_hbm.at[pl.ds(base + g * ch, ch)], buf, sem)

        def ld_wait(g, buf, sem):
            pltpu.make_async_copy(m_hbm.at[pl.ds(base + g * ch, ch)], buf,
                                  sem).wait()

        def ldi(g, ibuf):
            pltpu.sync_copy(dst_hbm.at[pl.ds(base + g * ch, ch)], ibuf)

        def sc(ibuf, buf, sem):
            pltpu.async_copy(buf, acc_sh.at[ibuf], sem, add=True)

        def sc_wait(ibuf, buf, sem):
            pltpu.make_async_copy(buf, acc_sh.at[ibuf], sem).wait()

        ldi(0, i0)
        ld(0, m0, ls0)

        def body(p, carry):
            c0 = 2 * p
            c1 = c0 + 1
            ld(c1, m1, ls1)
            ldi(c1, i1)
            ld_wait(c0, m0, ls0)
            pltpu.sync_copy(m0, acc_sh.at[i0], add=True)

            @pl.when(p < n_pair - 1)
            def _():
                ld(c0 + 2, m0, ls0)
                ldi(c0 + 2, i0)

            ld_wait(c1, m1, ls1)
            pltpu.sync_copy(m1, acc_sh.at[i1], add=True)
            return carry

        lax.fori_loop(0, n_pair, body, 0)
        for k_tail in range(2):
            g_tail = 2 * n_pair + k_tail

            @pl.when(g_tail < n_mine_e)
            def _():
                ldi(g_tail, i0)
                pltpu.sync_copy(m_hbm.at[pl.ds(base + g_tail * ch, ch)], m0)
                pltpu.sync_copy(m0, acc_sh.at[i0], add=True)

        plsc.subcore_barrier()

        def wchunk(k_, carry):
            r = (s + k_ * 16) * ch
            pltpu.sync_copy(acc_sh.at[pl.ds(r, ch)], m0)
            pltpu.sync_copy(m0, out_hbm.at[pl.ds(c * n_nodes + r, ch)])
            return carry

        lax.fori_loop(0, n_mine, wchunk, 0)

    return k(m, dst)


def _tc_edge(gathered, edge_attr, Ws, Wd, Wa, be1, We2, be2, bg, ge, be,
             Wm1b, bm1, Wm2, bm2, eoff):
    n_edges = gathered.shape[0] // 2
    blk = 1280
    nb = n_edges // blk
    bf = jnp.bfloat16

    def dot(a, b):
        return lax.dot_general(a, b, (((1,), (0,)), ((), ())),
                               preferred_element_type=jnp.float32)

    def body(hs_r, hd_r, ea_r, Ws_r, Wd_r, Wa_r, be1_r, We2_r, be2_r,
             bg_r, ge_r, be_r, Wm1b_r, bm1_r, Wm2_r, bm2_r, e_ref, m_ref):
        hs = hs_r[...].astype(bf)
        hd = hd_r[...].astype(bf)
        ea = ea_r[...]
        y_s = dot(hs, Ws_r[...])
        y_d = dot(hd, Wd_r[...])
        y_a = dot(ea.astype(bf), Wa_r[...])
        tg = y_s[:, HIDDEN:] + y_d + y_a
        t = _gelu(tg[:, :EDGE_DIM] + be1_r[...])
        gate = 0.5 * jnp.tanh((tg[:, EDGE_DIM:] + bg_r[...]) * 0.5) + 0.5
        x = ea + (dot(t.astype(bf), We2_r[...]) + be2_r[...]) * gate

        jmean = jnp.full((EDGE_DIM, EDGE_DIM), 1.0 / EDGE_DIM, bf)
        xc = x - dot(x.astype(bf), jmean)
        var = dot((xc * xc).astype(bf), jmean)
        e = xc * lax.rsqrt(var + 1e-5) * ge_r[...] + be_r[...]
        e_ref[...] = e
        u = _gelu(y_s[:, :HIDDEN] + dot(e.astype(bf), Wm1b_r[...]) + bm1_r[...])
        m_ref[...] = dot(u.astype(bf), Wm2_r[...]) + bm2_r[...]

    wspec = lambda shp: pl.BlockSpec(shp, lambda i: (0, 0))
    return pl.pallas_call(
        body,
        grid=(nb,),
        in_specs=[
            pl.BlockSpec((blk, HIDDEN), lambda i: (i, 0)),
            pl.BlockSpec((blk, HIDDEN), lambda i: (i + nb, 0)),
            pl.BlockSpec((blk, EDGE_DIM), lambda i: (i + eoff, 0)),
            wspec((HIDDEN, 145)), wspec((HIDDEN, 17)), wspec((EDGE_DIM, 17)),
            wspec((1, EDGE_DIM)),
            wspec((EDGE_DIM, EDGE_DIM)), wspec((1, EDGE_DIM)),
            wspec((1, 1)),
            wspec((1, EDGE_DIM)), wspec((1, EDGE_DIM)),
            wspec((EDGE_DIM, HIDDEN)), wspec((1, HIDDEN)),
            wspec((HIDDEN, HIDDEN)), wspec((1, HIDDEN)),
        ],
        out_specs=[
            pl.BlockSpec((blk, EDGE_DIM), lambda i: (i, 0)),
            pl.BlockSpec((blk, HIDDEN), lambda i: (i, 0)),
        ],
        out_shape=[
            jax.ShapeDtypeStruct((n_edges, EDGE_DIM), jnp.float32),
            jax.ShapeDtypeStruct((n_edges, HIDDEN), jnp.float32),
        ],
    )(gathered, gathered, edge_attr, Ws, Wd, Wa, be1, We2, be2,
      bg, ge, be, Wm1b, bm1, Wm2, bm2)


def _tc_node(h, parts_list, Wu1a, Wu1b, bu1, Wu2, bu2, gn, bn):
    n_nodes = h.shape[0]
    blk = 1000
    nb = n_nodes // blk
    np_ = len(parts_list)

    def dot(a, b):
        return lax.dot_general(a, b, (((1,), (0,)), ((), ())),
                               preferred_element_type=jnp.float32)

    bf = jnp.bfloat16

    def body(h_r, *rest):
        p_rs = rest[:2 * np_]
        Wu1a_r, Wu1b_r, bu1_r, Wu2_r, bu2_r, gn_r, bn_r, o_ref = rest[2 * np_:]
        hh = h_r[...]
        agg = p_rs[0][...]
        for p_r in p_rs[1:]:
            agg = agg + p_r[...]
        u = _gelu(dot(hh.astype(bf), Wu1a_r[...])
                  + dot(agg.astype(bf), Wu1b_r[...]) + bu1_r[...])
        h2 = dot(u.astype(bf), Wu2_r[...]) + bu2_r[...]
        x = hh + h2
        mu = jnp.mean(x, axis=-1, keepdims=True)
        var = jnp.mean((x - mu) ** 2, axis=-1, keepdims=True)
        o_ref[...] = (x - mu) / jnp.sqrt(var + 1e-5) * gn_r[...] + bn_r[...]

    wspec = lambda shp: pl.BlockSpec(shp, lambda i: (0, 0))
    part_specs = []
    part_args = []
    for parts in parts_list:
        part_specs += [pl.BlockSpec((blk, HIDDEN), lambda i: (i, 0)),
                       pl.BlockSpec((blk, HIDDEN), lambda i: (i + nb, 0))]
        part_args += [parts, parts]
    return pl.pallas_call(
        body,
        grid=(nb,),
        in_specs=[pl.BlockSpec((blk, HIDDEN), lambda i: (i, 0))] + part_specs + [
            wspec((HIDDEN, HIDDEN)), wspec((HIDDEN, HIDDEN)),
            wspec((1, HIDDEN)),
            wspec((HIDDEN, HIDDEN)), wspec((1, HIDDEN)),
            wspec((1, HIDDEN)), wspec((1, HIDDEN)),
        ],
        out_specs=pl.BlockSpec((blk, HIDDEN), lambda i: (i, 0)),
        out_shape=jax.ShapeDtypeStruct((n_nodes, HIDDEN), jnp.float32),
    )(h, *part_args, Wu1a, Wu1b, bu1, Wu2, bu2, gn, bn)


def kernel(h, edge_index, edge_attr, Wm1, bm1, Wm2, bm2, Wu1, bu1, Wu2, bu2,
           gn, bn, We1, be1, We2, be2, Wg, bg, ge, be):
    n_nodes = h.shape[0]
    H, D = HIDDEN, EDGE_DIM

    n_edges = edge_attr.shape[0]

    row = lambda v: v.reshape(1, -1)
    bf = jnp.bfloat16

    Ws = jnp.concatenate([Wm1[:H], We1[:H], Wg[:H]], axis=1).astype(bf)
    Wd = jnp.concatenate([We1[H:2 * H], Wg[H:2 * H]], axis=1).astype(bf)
    Wa = jnp.concatenate([We1[2 * H:], Wg[2 * H:]], axis=1).astype(bf)

    ns = 2
    es = n_edges // ns
    e_list, parts_list = [], []
    for j in range(ns):
        src_j = lax.dynamic_slice_in_dim(edge_index[0], j * es, es)
        dst_j = lax.dynamic_slice_in_dim(edge_index[1], j * es, es)
        idx_j = jnp.concatenate([src_j, dst_j])
        g_j = _sc_gather(h, idx_j)
        e_j, m_j = _tc_edge(
            g_j, edge_attr, Ws, Wd, Wa, row(be1),
            (EDGE_SCALE * We2).astype(bf), row(EDGE_SCALE * be2),
            row(bg), row(ge), row(be), Wm1[H:].astype(bf), row(bm1),
            Wm2.astype(bf), row(bm2), j * (es // 1280))
        parts_list.append(_sc_scatter(m_j, dst_j, n_nodes))
        e_list.append(e_j)
    e = jnp.concatenate(e_list, axis=0)
    hn = _tc_node(h, parts_list, Wu1[:H].astype(bf), Wu1[H:].astype(bf),
                  row(bu1), Wu2.astype(bf), row(bu2), row(gn), row(bn))
    return (hn, e)
